# Initial kernel scaffold; baseline (speedup 1.0000x reference)
import jax
import jax.numpy as jnp
from jax.experimental import pallas as pl


def kernel(hidden, y, centers):
    raise NotImplementedError("write your pallas kernel here")


# speedup vs baseline: 1.9284x; 1.9284x over previous
import jax
import jax.numpy as jnp
from jax.experimental import pallas as pl
from jax.experimental.pallas import tpu as pltpu

BATCH = 16384
D = 1024
K = 1024
BLK = 1024
NBLK = BATCH // BLK


def _body(y_ref, h_ref, c_ref, out_ref, acc_ref):
    i = pl.program_id(0)

    @pl.when(i == 0)
    def _():
        acc_ref[0, 0] = 0.0

    y_row = y_ref[0]
    ohT = (jax.lax.broadcasted_iota(jnp.int32, (K, BLK), 0) == y_row).astype(
        jnp.bfloat16
    )
    g = jax.lax.dot_general(
        ohT,
        c_ref[...],
        dimension_numbers=(((0,), (0,)), ((), ())),
        preferred_element_type=jnp.float32,
    )
    diff = h_ref[...] - g
    acc_ref[0, 0] += jnp.sum(diff * diff)

    @pl.when(i == pl.num_programs(0) - 1)
    def _():
        out_ref[0, 0] = jnp.sqrt(acc_ref[0, 0]) * (0.5 / BATCH)


def kernel(hidden, y, centers):
    y3 = y.astype(jnp.int32).reshape(NBLK, 1, BLK)
    c_bf = centers.astype(jnp.bfloat16)
    out = pl.pallas_call(
        _body,
        grid=(NBLK,),
        in_specs=[
            pl.BlockSpec((1, 1, BLK), lambda i: (i, 0, 0)),
            pl.BlockSpec((BLK, D), lambda i: (i, 0)),
            pl.BlockSpec((K, D), lambda i: (0, 0)),
        ],
        out_specs=pl.BlockSpec(memory_space=pltpu.SMEM),
        out_shape=jax.ShapeDtypeStruct((1, 1), jnp.float32),
        scratch_shapes=[pltpu.SMEM((1, 1), jnp.float32)],
    )(y3, hidden, c_bf)
    return out[0, 0]


# speedup vs baseline: 2.1243x; 1.1016x over previous
import jax
import jax.numpy as jnp
from jax.experimental import pallas as pl
from jax.experimental.pallas import tpu as pltpu

BATCH = 16384
D = 1024
K = 1024
BLK = 1024
NBLK = BATCH // BLK


def _body(y_ref, h_prev_ref, c_ref, ones_ref, out_ref, g_ref, acc_ref):
    i = pl.program_id(0)

    @pl.when(i == 0)
    def _():
        acc_ref[...] = jnp.zeros_like(acc_ref)

    diff = (h_prev_ref[...] - g_ref[...]).astype(jnp.bfloat16)
    dsq = diff * diff
    part = jax.lax.dot_general(
        ones_ref[...],
        dsq,
        dimension_numbers=(((1,), (0,)), ((), ())),
        preferred_element_type=jnp.float32,
    )
    acc_ref[...] += jnp.where(i > 0, part, jnp.zeros_like(part))

    y_row = y_ref[0]
    ohT = (
        jax.lax.broadcasted_iota(jnp.int32, (K, BLK), 0) == y_row
    ).astype(jnp.bfloat16)
    g_ref[...] = jax.lax.dot_general(
        ohT,
        c_ref[...],
        dimension_numbers=(((0,), (0,)), ((), ())),
        preferred_element_type=jnp.float32,
    )

    @pl.when(i == NBLK)
    def _():
        out_ref[0, 0] = jnp.sqrt(jnp.sum(acc_ref[...])) * (0.5 / BATCH)


def kernel(hidden, y, centers):
    y3 = y.astype(jnp.int32).reshape(NBLK, 1, BLK)
    c_bf = centers.astype(jnp.bfloat16)
    ones = jnp.ones((1, BLK), jnp.bfloat16)
    out = pl.pallas_call(
        _body,
        grid=(NBLK + 1,),
        in_specs=[
            pl.BlockSpec((1, 1, BLK), lambda i: (jnp.minimum(i, NBLK - 1), 0, 0)),
            pl.BlockSpec((BLK, D), lambda i: (jnp.maximum(i - 1, 0), 0)),
            pl.BlockSpec((K, D), lambda i: (0, 0)),
            pl.BlockSpec((1, BLK), lambda i: (0, 0)),
        ],
        out_specs=pl.BlockSpec(memory_space=pltpu.SMEM),
        out_shape=jax.ShapeDtypeStruct((1, 1), jnp.float32),
        scratch_shapes=[
            pltpu.VMEM((BLK, D), jnp.float32),
            pltpu.VMEM((1, D), jnp.float32),
        ],
    )(y3, hidden, c_bf, ones)
    return out[0, 0]
